# baseline (speedup 1.0000x reference)
import functools

import jax
import jax.numpy as jnp
from jax import lax
from jax.experimental import pallas as pl
from jax.experimental.pallas import tpu as pltpu
from jax.experimental.pallas import tpu_sc as plsc


def _node_pre_body(x_ref, w_ref, b_ref, a_ref, bc_ref, r_ref, *, out_c):
    p = jnp.dot(x_ref[...], w_ref[...], preferred_element_type=jnp.float32)
    p = p + b_ref[...]
    a_ref[...] = p[:, :out_c]
    bc_ref[...] = p[:, out_c:3 * out_c].astype(jnp.bfloat16)
    r_ref[...] = p[:, 3 * out_c:]


def _unpack_bf16(w):
    lo = lax.bitcast_convert_type(w << 16, jnp.float32)
    hi = lax.bitcast_convert_type(jnp.bitwise_and(w, jnp.int32(-65536)),
                                  jnp.float32)
    return jnp.concatenate([lo, hi], axis=1)


def _pack_bf16(m):
    k = m.shape[1] // 2

    def rnd(x):
        b = lax.bitcast_convert_type(x, jnp.int32)
        return lax.shift_right_logical(
            b + jnp.int32(0x7FFF)
            + jnp.bitwise_and(lax.shift_right_logical(b, 16), jnp.int32(1)),
            16)

    return jnp.bitwise_or(rnd(m[:, :k]),
                          lax.shift_left(rnd(m[:, k:]), jnp.int32(16)))


def _edge_body(gd_ref, gs_ref, ea_ref, w2_ref, b2_ref, we0_ref, msg_ref, *, out_c):
    gs = _unpack_bf16(gs_ref[...])
    u = jnp.maximum(gd_ref[...] + gs[:, :out_c], 0.0)
    hn = jnp.dot(u, w2_ref[...], preferred_element_type=jnp.float32) + b2_ref[...]
    he = jnp.dot(ea_ref[...], we0_ref[...], preferred_element_type=jnp.float32)
    he = jnp.maximum(he + gs[:, out_c:], 0.0)
    msg_ref[...] = _pack_bf16(hn * he)


def _final_body(*refs):
    agg_refs = refs[:-4]
    r_ref, g_ref, b_ref, o_ref = refs[-4:]
    s = r_ref[...]
    for a in agg_refs:
        s = s + a[0] + a[1]
    mean = jnp.mean(s, axis=0, keepdims=True)
    d = s - mean
    var = jnp.mean(d * d, axis=0, keepdims=True)
    o_ref[...] = jnp.maximum(
        d * lax.rsqrt(var + 1e-5) * g_ref[...] + b_ref[...], 0.0)


def _make_gather(n, e, out_c, nc, ns, ch, base0):
    npw = e // (nc * ns)
    mesh = plsc.VectorSubcoreMesh(core_axis_name="c", subcore_axis_name="s")

    nit = npw // ch
    assert nit >= 4

    @functools.partial(
        pl.kernel,
        out_type=(jax.ShapeDtypeStruct((e, out_c), jnp.float32),
                  jax.ShapeDtypeStruct((e, out_c), jnp.int32)),
        mesh=mesh,
        scratch_types=[
            pltpu.VMEM((2, ch), jnp.int32),
            pltpu.VMEM((2, ch), jnp.int32),
            pltpu.VMEM((2, ch, out_c), jnp.float32),
            pltpu.VMEM((2, ch, out_c), jnp.int32),
            pltpu.SemaphoreType.DMA,
            pltpu.SemaphoreType.DMA,
            pltpu.SemaphoreType.DMA,
            pltpu.SemaphoreType.DMA,
        ],
    )
    def gather_k(a_hbm, bc_hbm, src_hbm, dst_hbm, gd_hbm, gs_hbm,
                 idx_s, idx_d, gd_v, gs_v, sg_a, sg_b, sw_a, sw_b):
        wid = lax.axis_index("s") * nc + lax.axis_index("c")
        base = wid * npw
        sg = (sg_a, sg_b)
        sw = (sw_a, sw_b)

        def idx_copy(i, b):
            off = base0 + base + i * ch
            pltpu.sync_copy(src_hbm.at[pl.ds(off, ch)], idx_s.at[b])
            pltpu.sync_copy(dst_hbm.at[pl.ds(off, ch)], idx_d.at[b])

        def g_issue(b):
            pltpu.async_copy(a_hbm.at[idx_d.at[b]], gd_v.at[b], sg[b])
            pltpu.async_copy(bc_hbm.at[idx_s.at[b]], gs_v.at[b], sg[b])

        def g_wait(b):
            pltpu.make_async_copy(a_hbm.at[idx_d.at[b]], gd_v.at[b], sg[b]).wait()
            pltpu.make_async_copy(bc_hbm.at[idx_s.at[b]], gs_v.at[b], sg[b]).wait()

        def wb_issue(i, b):
            off = base + i * ch
            pltpu.async_copy(gd_v.at[b], gd_hbm.at[pl.ds(off, ch)], sw[b])
            pltpu.async_copy(gs_v.at[b], gs_hbm.at[pl.ds(off, ch)], sw[b])

        def wb_wait(i, b):
            off = base + i * ch
            pltpu.make_async_copy(gd_v.at[b], gd_hbm.at[pl.ds(off, ch)], sw[b]).wait()
            pltpu.make_async_copy(gs_v.at[b], gs_hbm.at[pl.ds(off, ch)], sw[b]).wait()

        idx_copy(0, 0)
        g_issue(0)

        def body(k, carry):
            i0 = 2 * k
            idx_copy(i0 + 1, 1)

            @pl.when(k > 0)
            def _():
                wb_wait(i0 - 1, 1)
            g_issue(1)
            g_wait(0)
            wb_issue(i0, 0)
            idx_copy(i0 + 2, 0)
            wb_wait(i0, 0)
            g_issue(0)
            g_wait(1)
            wb_issue(i0 + 1, 1)
            return carry

        if nit % 2 == 1:
            lax.fori_loop(0, (nit - 1) // 2, body, 0)
            g_wait(0)
            wb_issue(nit - 1, 0)
            wb_wait(nit - 2, 1)
            wb_wait(nit - 1, 0)
        else:
            lax.fori_loop(0, nit // 2 - 1, body, 0)
            idx_copy(nit - 1, 1)
            wb_wait(nit - 3, 1)
            g_issue(1)
            g_wait(0)
            wb_issue(nit - 2, 0)
            g_wait(1)
            wb_issue(nit - 1, 1)
            wb_wait(nit - 2, 0)
            wb_wait(nit - 1, 1)

    return gather_k


def _make_scatter(n, e, out_c, nc, ns, ch, base0):
    npw = e // (nc * ns)
    zch = 80
    nchunks = n // zch
    jmax = -(-nchunks // ns)
    mesh = plsc.VectorSubcoreMesh(core_axis_name="c", subcore_axis_name="s")

    nit = npw // ch
    assert nit >= 4

    @functools.partial(
        pl.kernel,
        out_type=jax.ShapeDtypeStruct((2, n, out_c), jnp.float32),
        mesh=mesh,
        scratch_types=[
            pltpu.VMEM((2, ch), jnp.int32),
            pltpu.VMEM((2, ch, out_c // 2), jnp.int32),
            pltpu.VMEM((ch,), jnp.int32),
            pltpu.VMEM((ch,), jnp.int32),
            pltpu.VMEM((ch, out_c), jnp.float32),
            pltpu.VMEM((ch, out_c), jnp.float32),
            pltpu.VMEM_SHARED((n, out_c), jnp.float32),
            pltpu.SemaphoreType.DMA,
            pltpu.SemaphoreType.DMA,
            pltpu.SemaphoreType.DMA,
            pltpu.SemaphoreType.DMA,
        ],
    )
    def scatter_k(msg_hbm, dst_hbm, zeros_hbm, agg_hbm,
                  idx_d, m_pk, si_a, si_b, m32_a, m32_b, acc_sh,
                  sp_a, sp_b, ss_a, ss_b):
        c = lax.axis_index("c")
        s = lax.axis_index("s")
        wid = s * nc + c
        base = wid * npw
        sp = (sp_a, sp_b)
        ss = (ss_a, ss_b)
        si = (si_a, si_b)
        m32 = (m32_a, m32_b)

        def pf_issue(i, b):
            off = base + i * ch
            pltpu.async_copy(dst_hbm.at[pl.ds(base0 + off, ch)], idx_d.at[b], sp[b])
            pltpu.async_copy(msg_hbm.at[pl.ds(off, ch)], m_pk.at[b], sp[b])

        def pf_wait(i, b):
            off = base + i * ch
            pltpu.make_async_copy(dst_hbm.at[pl.ds(base0 + off, ch)], idx_d.at[b], sp[b]).wait()
            pltpu.make_async_copy(msg_hbm.at[pl.ds(off, ch)], m_pk.at[b], sp[b]).wait()

        def scat_issue(b):
            pltpu.async_copy(m32[b], acc_sh.at[si[b]], ss[b], add=True)

        def scat_wait(b):
            pltpu.make_async_copy(m32[b], acc_sh.at[si[b]], ss[b]).wait()

        def phase(i, b):
            o = 1 - b

            @pl.when(i + 1 < nit)
            def _pf():
                pf_issue(i + 1, o)
            pf_wait(i, b)

            @pl.when(i >= 2)
            def _sw():
                scat_wait(b)
            for j in range(ch // 16):
                si[b][pl.ds(j * 16, 16)] = idx_d[b, pl.ds(j * 16, 16)]

            def urow(r4, carry):
                for rr in range(4):
                    r = r4 * 4 + rr
                    for j in range(out_c // 32):
                        w = m_pk[b, r, pl.ds(j * 16, 16)]
                        lo = lax.bitcast_convert_type(
                            lax.shift_left(w, jnp.int32(16)), jnp.float32)
                        hi = lax.bitcast_convert_type(
                            jnp.bitwise_and(w, jnp.int32(-65536)), jnp.float32)
                        m32[b][r, pl.ds(j * 16, 16)] = lo
                        m32[b][r, pl.ds(out_c // 2 + j * 16, 16)] = hi
                return carry

            lax.fori_loop(0, ch // 4, urow, 0)
            scat_issue(b)

        pf_issue(0, 0)
        pltpu.sync_copy(zeros_hbm.at[pl.ds(0, zch)], m32_a)
        for j in range(jmax):
            cid = j * ns + s

            @pl.when(cid < nchunks)
            def _zero():
                pltpu.sync_copy(m32_a, acc_sh.at[pl.ds(cid * zch, zch)])
        plsc.subcore_barrier()

        def body(k, carry):
            phase(2 * k, 0)

            @pl.when(2 * k + 1 < nit)
            def _odd():
                phase(2 * k + 1, 1)
            return carry

        lax.fori_loop(0, (nit + 1) // 2, body, 0)
        scat_wait((nit - 2) % 2)
        scat_wait((nit - 1) % 2)
        plsc.subcore_barrier()

        for j in range(jmax):
            cid = j * ns + s

            @pl.when(cid < nchunks)
            def _wb():
                pltpu.sync_copy(acc_sh.at[pl.ds(cid * zch, zch)],
                                agg_hbm.at[c, pl.ds(cid * zch, zch)])

    return scatter_k


def kernel(x, edge_index, edge_attr, W1, b1, W2, b2, We, be, Wr, br,
           gamma, beta):
    n, d = x.shape
    e = edge_index.shape[1]
    de = edge_attr.shape[1]
    out_c = W2.shape[1]

    info = plsc.get_sparse_core_info()
    nc, ns = info.num_cores, info.num_subcores
    ch = 80

    src = edge_index[0]
    dst = edge_index[1]

    w_cat = jnp.concatenate([W1[:d], W1[d:], We[de:], Wr], axis=1)
    b_cat = jnp.concatenate(
        [b1, jnp.zeros((out_c,), jnp.float32), be, br])[None, :]

    blk_n = 1000
    a_tab, bc_tab, r_tab = pl.pallas_call(
        functools.partial(_node_pre_body, out_c=out_c),
        grid=(n // blk_n,),
        in_specs=[
            pl.BlockSpec((blk_n, d), lambda i: (i, 0)),
            pl.BlockSpec((d, 4 * out_c), lambda i: (0, 0)),
            pl.BlockSpec((1, 4 * out_c), lambda i: (0, 0)),
        ],
        out_specs=[
            pl.BlockSpec((blk_n, out_c), lambda i: (i, 0)),
            pl.BlockSpec((blk_n, 2 * out_c), lambda i: (i, 0)),
            pl.BlockSpec((blk_n, out_c), lambda i: (i, 0)),
        ],
        out_shape=[
            jax.ShapeDtypeStruct((n, out_c), jnp.float32),
            jax.ShapeDtypeStruct((n, 2 * out_c), jnp.bfloat16),
            jax.ShapeDtypeStruct((n, out_c), jnp.float32),
        ],
    )(x, w_cat, b_cat)

    bc_pk = lax.bitcast_convert_type(
        jnp.stack([bc_tab[:, :out_c], bc_tab[:, out_c:]], axis=-1),
        jnp.int32)

    blk_e = 1280
    unit = nc * ns * ch
    n_units = e // unit
    n_slab = 3
    per = [n_units // n_slab] * n_slab
    per[0] += n_units - sum(per)
    zeros = jnp.zeros((n, out_c), jnp.float32)
    parts = []
    base0 = 0
    for k in range(n_slab):
        es = per[k] * unit
        gd, gs = _make_gather(n, es, out_c, nc, ns, ch, base0)(
            a_tab, bc_pk, src, dst)
        off_blk = base0 // blk_e
        msg = pl.pallas_call(
            functools.partial(_edge_body, out_c=out_c),
            grid=(es // blk_e,),
            in_specs=[
                pl.BlockSpec((blk_e, out_c), lambda i: (i, 0)),
                pl.BlockSpec((blk_e, out_c), lambda i: (i, 0)),
                pl.BlockSpec((blk_e, de), lambda i, o=off_blk: (i + o, 0)),
                pl.BlockSpec((d, out_c), lambda i: (0, 0)),
                pl.BlockSpec((1, out_c), lambda i: (0, 0)),
                pl.BlockSpec((de, out_c), lambda i: (0, 0)),
            ],
            out_specs=pl.BlockSpec((blk_e, out_c // 2), lambda i: (i, 0)),
            out_shape=jax.ShapeDtypeStruct((es, out_c // 2), jnp.int32),
        )(gd, gs, edge_attr, W2, b2[None, :], We[:de])
        parts.append(
            _make_scatter(n, es, out_c, nc, ns, ch, base0)(msg, dst, zeros))
        base0 += es

    out = pl.pallas_call(
        _final_body,
        out_shape=jax.ShapeDtypeStruct((n, out_c), jnp.float32),
    )(*parts, r_tab, gamma[None, :], beta[None, :])

    return (out, edge_index, edge_attr)


# speedup vs baseline: 1.0239x; 1.0239x over previous
import functools

import jax
import jax.numpy as jnp
from jax import lax
from jax.experimental import pallas as pl
from jax.experimental.pallas import tpu as pltpu
from jax.experimental.pallas import tpu_sc as plsc


def _node_pre_body(x_ref, w_ref, b_ref, a_ref, bc_ref, r_ref, *, out_c):
    p = jnp.dot(x_ref[...], w_ref[...], preferred_element_type=jnp.float32)
    p = p + b_ref[...]
    a_ref[...] = p[:, :out_c]
    bc_ref[...] = p[:, out_c:3 * out_c].astype(jnp.bfloat16)
    r_ref[...] = p[:, 3 * out_c:]


def _unpack_bf16(w):
    lo = lax.bitcast_convert_type(w << 16, jnp.float32)
    hi = lax.bitcast_convert_type(jnp.bitwise_and(w, jnp.int32(-65536)),
                                  jnp.float32)
    return jnp.concatenate([lo, hi], axis=1)


def _edge_body(gd_ref, gs_ref, ea_ref, w2_ref, b2_ref, we0_ref, msg_ref, *, out_c):
    gs = _unpack_bf16(gs_ref[...])
    u = jnp.maximum(gd_ref[...] + gs[:, :out_c], 0.0)
    hn = jnp.dot(u, w2_ref[...], preferred_element_type=jnp.float32) + b2_ref[...]
    he = jnp.dot(ea_ref[...], we0_ref[...], preferred_element_type=jnp.float32)
    he = jnp.maximum(he + gs[:, out_c:], 0.0)
    msg_ref[...] = hn * he


def _final_body(*refs):
    agg_refs = refs[:-4]
    r_ref, g_ref, b_ref, o_ref = refs[-4:]
    s = r_ref[...]
    for a in agg_refs:
        s = s + a[0] + a[1]
    mean = jnp.mean(s, axis=0, keepdims=True)
    d = s - mean
    var = jnp.mean(d * d, axis=0, keepdims=True)
    o_ref[...] = jnp.maximum(
        d * lax.rsqrt(var + 1e-5) * g_ref[...] + b_ref[...], 0.0)


def _make_gather(n, e, out_c, nc, ns, ch, base0):
    npw = e // (nc * ns)
    mesh = plsc.VectorSubcoreMesh(core_axis_name="c", subcore_axis_name="s")

    nit = npw // ch
    assert nit >= 4

    @functools.partial(
        pl.kernel,
        out_type=(jax.ShapeDtypeStruct((e, out_c), jnp.float32),
                  jax.ShapeDtypeStruct((e, out_c), jnp.int32)),
        mesh=mesh,
        scratch_types=[
            pltpu.VMEM((2, ch), jnp.int32),
            pltpu.VMEM((2, ch), jnp.int32),
            pltpu.VMEM((2, ch, out_c), jnp.float32),
            pltpu.VMEM((2, ch, out_c), jnp.int32),
            pltpu.SemaphoreType.DMA,
            pltpu.SemaphoreType.DMA,
            pltpu.SemaphoreType.DMA,
            pltpu.SemaphoreType.DMA,
        ],
    )
    def gather_k(a_hbm, bc_hbm, src_hbm, dst_hbm, gd_hbm, gs_hbm,
                 idx_s, idx_d, gd_v, gs_v, sg_a, sg_b, sw_a, sw_b):
        wid = lax.axis_index("s") * nc + lax.axis_index("c")
        base = wid * npw
        sg = (sg_a, sg_b)
        sw = (sw_a, sw_b)

        def idx_copy(i, b):
            off = base0 + base + i * ch
            pltpu.sync_copy(src_hbm.at[pl.ds(off, ch)], idx_s.at[b])
            pltpu.sync_copy(dst_hbm.at[pl.ds(off, ch)], idx_d.at[b])

        def g_issue(b):
            pltpu.async_copy(a_hbm.at[idx_d.at[b]], gd_v.at[b], sg[b])
            pltpu.async_copy(bc_hbm.at[idx_s.at[b]], gs_v.at[b], sg[b])

        def g_wait(b):
            pltpu.make_async_copy(a_hbm.at[idx_d.at[b]], gd_v.at[b], sg[b]).wait()
            pltpu.make_async_copy(bc_hbm.at[idx_s.at[b]], gs_v.at[b], sg[b]).wait()

        def wb_issue(i, b):
            off = base + i * ch
            pltpu.async_copy(gd_v.at[b], gd_hbm.at[pl.ds(off, ch)], sw[b])
            pltpu.async_copy(gs_v.at[b], gs_hbm.at[pl.ds(off, ch)], sw[b])

        def wb_wait(i, b):
            off = base + i * ch
            pltpu.make_async_copy(gd_v.at[b], gd_hbm.at[pl.ds(off, ch)], sw[b]).wait()
            pltpu.make_async_copy(gs_v.at[b], gs_hbm.at[pl.ds(off, ch)], sw[b]).wait()

        idx_copy(0, 0)
        g_issue(0)

        def body(k, carry):
            i0 = 2 * k
            idx_copy(i0 + 1, 1)

            @pl.when(k > 0)
            def _():
                wb_wait(i0 - 1, 1)
            g_issue(1)
            g_wait(0)
            wb_issue(i0, 0)
            idx_copy(i0 + 2, 0)
            wb_wait(i0, 0)
            g_issue(0)
            g_wait(1)
            wb_issue(i0 + 1, 1)
            return carry

        if nit % 2 == 1:
            lax.fori_loop(0, (nit - 1) // 2, body, 0)
            g_wait(0)
            wb_issue(nit - 1, 0)
            wb_wait(nit - 2, 1)
            wb_wait(nit - 1, 0)
        else:
            lax.fori_loop(0, nit // 2 - 1, body, 0)
            idx_copy(nit - 1, 1)
            wb_wait(nit - 3, 1)
            g_issue(1)
            g_wait(0)
            wb_issue(nit - 2, 0)
            g_wait(1)
            wb_issue(nit - 1, 1)
            wb_wait(nit - 2, 0)
            wb_wait(nit - 1, 1)

    return gather_k


def _make_scatter(n, e, out_c, nc, ns, ch, base0):
    npw = e // (nc * ns)
    zch = 80
    nchunks = n // zch
    jmax = -(-nchunks // ns)
    mesh = plsc.VectorSubcoreMesh(core_axis_name="c", subcore_axis_name="s")

    nit = npw // ch
    assert nit >= 4

    @functools.partial(
        pl.kernel,
        out_type=jax.ShapeDtypeStruct((2, n, out_c), jnp.float32),
        mesh=mesh,
        scratch_types=[
            pltpu.VMEM((2, ch), jnp.int32),
            pltpu.VMEM((2, ch, out_c), jnp.float32),
            pltpu.VMEM((zch, out_c), jnp.float32),
            pltpu.VMEM_SHARED((n, out_c), jnp.float32),
            pltpu.SemaphoreType.DMA,
            pltpu.SemaphoreType.DMA,
        ],
    )
    def scatter_k(msg_hbm, dst_hbm, zeros_hbm, agg_hbm,
                  idx_d, m_v, zbuf, acc_sh, sp_a, sp_b):
        c = lax.axis_index("c")
        s = lax.axis_index("s")
        wid = s * nc + c
        base = wid * npw
        sp = (sp_a, sp_b)

        def pf_issue(i, b):
            off = base + i * ch
            pltpu.async_copy(dst_hbm.at[pl.ds(base0 + off, ch)], idx_d.at[b], sp[b])
            pltpu.async_copy(msg_hbm.at[pl.ds(off, ch)], m_v.at[b], sp[b])

        def pf_wait(i, b):
            off = base + i * ch
            pltpu.make_async_copy(dst_hbm.at[pl.ds(base0 + off, ch)], idx_d.at[b], sp[b]).wait()
            pltpu.make_async_copy(msg_hbm.at[pl.ds(off, ch)], m_v.at[b], sp[b]).wait()

        def scat(b):
            pltpu.sync_copy(m_v.at[b], acc_sh.at[idx_d.at[b]], add=True)

        pf_issue(0, 0)
        pltpu.sync_copy(zeros_hbm.at[pl.ds(0, zch)], zbuf)
        for j in range(jmax):
            cid = j * ns + s

            @pl.when(cid < nchunks)
            def _zero():
                pltpu.sync_copy(zbuf, acc_sh.at[pl.ds(cid * zch, zch)])
        plsc.subcore_barrier()

        def body(k, carry):
            i0 = 2 * k
            pf_issue(i0 + 1, 1)
            pf_wait(i0, 0)
            scat(0)
            pf_issue(i0 + 2, 0)
            pf_wait(i0 + 1, 1)
            scat(1)
            return carry

        if nit % 2 == 1:
            lax.fori_loop(0, (nit - 1) // 2, body, 0)
            pf_wait(nit - 1, 0)
            scat(0)
        else:
            lax.fori_loop(0, nit // 2 - 1, body, 0)
            pf_issue(nit - 1, 1)
            pf_wait(nit - 2, 0)
            scat(0)
            pf_wait(nit - 1, 1)
            scat(1)
        plsc.subcore_barrier()

        for j in range(jmax):
            cid = j * ns + s

            @pl.when(cid < nchunks)
            def _wb():
                pltpu.sync_copy(acc_sh.at[pl.ds(cid * zch, zch)],
                                agg_hbm.at[c, pl.ds(cid * zch, zch)])

    return scatter_k


def kernel(x, edge_index, edge_attr, W1, b1, W2, b2, We, be, Wr, br,
           gamma, beta):
    n, d = x.shape
    e = edge_index.shape[1]
    de = edge_attr.shape[1]
    out_c = W2.shape[1]

    info = plsc.get_sparse_core_info()
    nc, ns = info.num_cores, info.num_subcores
    ch = 80

    src = edge_index[0]
    dst = edge_index[1]

    w_cat = jnp.concatenate([W1[:d], W1[d:], We[de:], Wr], axis=1)
    b_cat = jnp.concatenate(
        [b1, jnp.zeros((out_c,), jnp.float32), be, br])[None, :]

    blk_n = 1000
    a_tab, bc_tab, r_tab = pl.pallas_call(
        functools.partial(_node_pre_body, out_c=out_c),
        grid=(n // blk_n,),
        in_specs=[
            pl.BlockSpec((blk_n, d), lambda i: (i, 0)),
            pl.BlockSpec((d, 4 * out_c), lambda i: (0, 0)),
            pl.BlockSpec((1, 4 * out_c), lambda i: (0, 0)),
        ],
        out_specs=[
            pl.BlockSpec((blk_n, out_c), lambda i: (i, 0)),
            pl.BlockSpec((blk_n, 2 * out_c), lambda i: (i, 0)),
            pl.BlockSpec((blk_n, out_c), lambda i: (i, 0)),
        ],
        out_shape=[
            jax.ShapeDtypeStruct((n, out_c), jnp.float32),
            jax.ShapeDtypeStruct((n, 2 * out_c), jnp.bfloat16),
            jax.ShapeDtypeStruct((n, out_c), jnp.float32),
        ],
    )(x, w_cat, b_cat)

    bc_pk = lax.bitcast_convert_type(
        jnp.stack([bc_tab[:, :out_c], bc_tab[:, out_c:]], axis=-1),
        jnp.int32)

    blk_e = 1280
    unit = nc * ns * ch
    n_units = e // unit
    n_slab = 3
    per = [n_units // n_slab] * n_slab
    per[0] += n_units - sum(per)
    zeros = jnp.zeros((n, out_c), jnp.float32)
    parts = []
    base0 = 0
    for k in range(n_slab):
        es = per[k] * unit
        gd, gs = _make_gather(n, es, out_c, nc, ns, ch, base0)(
            a_tab, bc_pk, src, dst)
        off_blk = base0 // blk_e
        msg = pl.pallas_call(
            functools.partial(_edge_body, out_c=out_c),
            grid=(es // blk_e,),
            in_specs=[
                pl.BlockSpec((blk_e, out_c), lambda i: (i, 0)),
                pl.BlockSpec((blk_e, out_c), lambda i: (i, 0)),
                pl.BlockSpec((blk_e, de), lambda i, o=off_blk: (i + o, 0)),
                pl.BlockSpec((d, out_c), lambda i: (0, 0)),
                pl.BlockSpec((1, out_c), lambda i: (0, 0)),
                pl.BlockSpec((de, out_c), lambda i: (0, 0)),
            ],
            out_specs=pl.BlockSpec((blk_e, out_c), lambda i: (i, 0)),
            out_shape=jax.ShapeDtypeStruct((es, out_c), jnp.float32),
        )(gd, gs, edge_attr, W2, b2[None, :], We[:de])
        parts.append(
            _make_scatter(n, es, out_c, nc, ns, ch, base0)(msg, dst, zeros))
        base0 += es

    out = pl.pallas_call(
        _final_body,
        out_shape=jax.ShapeDtypeStruct((n, out_c), jnp.float32),
    )(*parts, r_tab, gamma[None, :], beta[None, :])

    return (out, edge_index, edge_attr)
